# baseline (speedup 1.0000x reference)
import functools

import jax
import jax.numpy as jnp
from jax import lax
from jax.experimental import pallas as pl
from jax.experimental.pallas import tpu as pltpu
from jax.experimental.pallas import tpu_sc as plsc

N_NODES = 100000
D_FEAT = 128
BATCH = 8192
LANES = 16
CHUNKS = (64, 128, 64)


@functools.lru_cache(maxsize=None)
def _build(batch, d_feat):
    info = plsc.get_sparse_core_info()
    nw = info.num_cores * info.num_subcores
    b_per_w = batch // nw
    assert sum(CHUNKS) == b_per_w
    offs = [sum(CHUNKS[:j]) for j in range(len(CHUNKS))]
    mesh = plsc.VectorSubcoreMesh(core_axis_name="c", subcore_axis_name="s")

    @functools.partial(
        pl.kernel,
        mesh=mesh,
        out_type=jax.ShapeDtypeStruct((batch, d_feat), jnp.float32),
        scratch_types=[
            pltpu.VMEM((b_per_w,), jnp.int32),
            pltpu.VMEM((b_per_w, d_feat), jnp.float32),
            pltpu.SemaphoreType.DMA,
            pltpu.SemaphoreType.DMA,
            pltpu.SemaphoreType.DMA,
            pltpu.SemaphoreType.DMA,
        ],
    )
    def gather_add(nf_hbm, mem_hbm, idx_hbm, out_hbm, idx_v, rows, sem_i, sem_a, sem_b, sem_o):
        wid = lax.axis_index("s") * info.num_cores + lax.axis_index("c")
        base = wid * b_per_w
        sls = [pl.ds(o, c) for o, c in zip(offs, CHUNKS)]
        idx_copies = [
            pltpu.async_copy(idx_hbm.at[pl.ds(base + o, c)], idx_v.at[sl], sem_i)
            for o, c, sl in zip(offs, CHUNKS, sls)
        ]
        nf_copies = []
        for j, sl in enumerate(sls):
            idx_copies[j].wait()
            nf_copies.append(
                pltpu.async_copy(nf_hbm.at[idx_v.at[sl]], rows.at[sl], sem_a)
            )
        add_copies = []
        for j, sl in enumerate(sls):
            nf_copies[j].wait()
            add_copies.append(
                pltpu.async_copy(mem_hbm.at[idx_v.at[sl]], rows.at[sl], sem_b, add=True)
            )
        out_copies = []
        for j, sl in enumerate(sls):
            add_copies[j].wait()
            out_copies.append(
                pltpu.async_copy(
                    rows.at[sl], out_hbm.at[pl.ds(base + offs[j], CHUNKS[j])], sem_o
                )
            )
        for c in out_copies:
            c.wait()

    return gather_add


def kernel(node_features, memory, source_nodes, timestamps, time_w, time_b):
    del timestamps, time_w, time_b
    idx = source_nodes.astype(jnp.int32)
    return _build(idx.shape[0], node_features.shape[1])(node_features, memory, idx)


# speedup vs baseline: 1.0098x; 1.0098x over previous
import functools

import jax
import jax.numpy as jnp
from jax import lax
from jax.experimental import pallas as pl
from jax.experimental.pallas import tpu as pltpu
from jax.experimental.pallas import tpu_sc as plsc

N_NODES = 100000
D_FEAT = 128
BATCH = 8192
LANES = 16
CHUNKS = (64, 64, 64, 64)


@functools.lru_cache(maxsize=None)
def _build(batch, d_feat):
    info = plsc.get_sparse_core_info()
    nw = info.num_cores * info.num_subcores
    b_per_w = batch // nw
    assert sum(CHUNKS) == b_per_w
    offs = [sum(CHUNKS[:j]) for j in range(len(CHUNKS))]
    mesh = plsc.VectorSubcoreMesh(core_axis_name="c", subcore_axis_name="s")

    @functools.partial(
        pl.kernel,
        mesh=mesh,
        out_type=jax.ShapeDtypeStruct((batch, d_feat), jnp.float32),
        scratch_types=[
            pltpu.VMEM((b_per_w,), jnp.int32),
            pltpu.VMEM((b_per_w, d_feat), jnp.float32),
            pltpu.SemaphoreType.DMA,
            pltpu.SemaphoreType.DMA,
            pltpu.SemaphoreType.DMA,
            pltpu.SemaphoreType.DMA,
        ],
    )
    def gather_add(nf_hbm, mem_hbm, idx_hbm, out_hbm, idx_v, rows, sem_i, sem_a, sem_b, sem_o):
        wid = lax.axis_index("s") * info.num_cores + lax.axis_index("c")
        base = wid * b_per_w
        sls = [pl.ds(o, c) for o, c in zip(offs, CHUNKS)]
        idx_copies = [
            pltpu.async_copy(idx_hbm.at[pl.ds(base + o, c)], idx_v.at[sl], sem_i)
            for o, c, sl in zip(offs, CHUNKS, sls)
        ]
        nf_copies = []
        for j, sl in enumerate(sls):
            idx_copies[j].wait()
            nf_copies.append(
                pltpu.async_copy(nf_hbm.at[idx_v.at[sl]], rows.at[sl], sem_a)
            )
        add_copies = []
        for j, sl in enumerate(sls):
            nf_copies[j].wait()
            add_copies.append(
                pltpu.async_copy(mem_hbm.at[idx_v.at[sl]], rows.at[sl], sem_b, add=True)
            )
        out_copies = []
        for j, sl in enumerate(sls):
            add_copies[j].wait()
            out_copies.append(
                pltpu.async_copy(
                    rows.at[sl], out_hbm.at[pl.ds(base + offs[j], CHUNKS[j])], sem_o
                )
            )
        for c in out_copies:
            c.wait()

    return gather_add


def kernel(node_features, memory, source_nodes, timestamps, time_w, time_b):
    del timestamps, time_w, time_b
    idx = source_nodes.astype(jnp.int32)
    return _build(idx.shape[0], node_features.shape[1])(node_features, memory, idx)
